# Initial kernel scaffold; baseline (speedup 1.0000x reference)
import jax
import jax.numpy as jnp
from jax.experimental import pallas as pl


def kernel(in_feat, edge_index, W_embed, b_embed, W_self1, W_neigh1, b1, W_self2, W_neigh2, b2, W_gat, attn_l, attn_r, b_gat, W_fc1, b_fc1, W_fc2, b_fc2):
    raise NotImplementedError("write your pallas kernel here")


# speedup vs baseline: 8.7462x; 8.7462x over previous
import functools

import jax
import jax.numpy as jnp
from jax import lax
from jax.experimental import pallas as pl
from jax.experimental.pallas import tpu as pltpu
from jax.experimental.pallas import tpu_sc as plsc

N = 10000
E = 320000
H = 128
HEADS = 4
C = 40

NC = 2
NS = 16
NW = NC * NS
EPW = E // NW
CH = 80
NCH = EPW // CH
NP = 10240
SR = NP // NS
F32 = jnp.float32
I32 = jnp.int32

BN = 2000
NB = N // BN

_MESH = plsc.VectorSubcoreMesh(core_axis_name="c", subcore_axis_name="s",
                               num_cores=NC, num_subcores=NS)
_SC_PARAMS = pltpu.CompilerParams(needs_layout_passes=False)


def _zero_rows(rows_v, width):
    zero16 = jnp.zeros((16,), F32)

    def zrow(i, _):
        for f in range(width // 16):
            rows_v[i, pl.ds(f * 16, 16)] = zero16
        return 0
    lax.fori_loop(0, CH, zrow, 0)


def _zero_acc(rows_v, acc_sh, s):
    for k in range(SR // CH):
        pltpu.sync_copy(rows_v, acc_sh.at[pl.ds(s * SR + k * CH, CH)])


def _seg_body(x_hbm, src_hbm, dst_hbm, out_hbm,
              rows_v, sidx_v, didx_v, acc_sh, sem):
    c = lax.axis_index("c")
    s = lax.axis_index("s")
    wid = c * NS + s
    _zero_rows(rows_v, H)
    _zero_acc(rows_v, acc_sh, s)
    plsc.subcore_barrier()

    def chunk(cix, _):
        e0 = wid * EPW + cix * CH
        pltpu.sync_copy(src_hbm.at[pl.ds(e0, CH)], sidx_v)
        pltpu.sync_copy(dst_hbm.at[pl.ds(e0, CH)], didx_v)
        pltpu.async_copy(x_hbm.at[sidx_v], rows_v, sem).wait()
        pltpu.sync_copy(rows_v, acc_sh.at[didx_v], add=True)
        return 0
    lax.fori_loop(0, NCH, chunk, 0)
    plsc.subcore_barrier()

    r0 = s * SR
    pltpu.sync_copy(acc_sh.at[pl.ds(r0, SR)], out_hbm.at[pl.ds(c * NP + r0, SR)])


_seg_sum = pl.kernel(
    _seg_body,
    out_type=jax.ShapeDtypeStruct((NC * NP, H), F32),
    mesh=_MESH,
    scratch_types=[
        pltpu.VMEM((CH, H), F32),
        pltpu.VMEM((CH,), I32),
        pltpu.VMEM((CH,), I32),
        pltpu.VMEM_SHARED((NP, H), F32),
        pltpu.SemaphoreType.DMA,
    ],
)


def _deg_body(dst_hbm, out_hbm, rows_v, didx_v, acc_sh):
    c = lax.axis_index("c")
    s = lax.axis_index("s")
    wid = c * NS + s
    _zero_rows(rows_v, H)
    _zero_acc(rows_v, acc_sh, s)
    one16 = jnp.ones((16,), F32)

    def orow(i, _):
        rows_v[i, pl.ds(0, 16)] = one16
        return 0
    lax.fori_loop(0, CH, orow, 0)
    plsc.subcore_barrier()

    def chunk(cix, _):
        e0 = wid * EPW + cix * CH
        pltpu.sync_copy(dst_hbm.at[pl.ds(e0, CH)], didx_v)
        pltpu.sync_copy(rows_v, acc_sh.at[didx_v], add=True)
        return 0
    lax.fori_loop(0, NCH, chunk, 0)
    plsc.subcore_barrier()

    r0 = s * SR
    pltpu.sync_copy(acc_sh.at[pl.ds(r0, SR)], out_hbm.at[pl.ds(c * NP + r0, SR)])


_deg_kernel = pl.kernel(
    _deg_body,
    out_type=jax.ShapeDtypeStruct((NC * NP, H), F32),
    mesh=_MESH,
    scratch_types=[
        pltpu.VMEM((CH, H), F32),
        pltpu.VMEM((CH,), I32),
        pltpu.VMEM_SHARED((NP, H), F32),
    ],
)


def _t_body(el_hbm, er_hbm, src_hbm, dst_hbm, t_hbm,
            el_v, er_v, tp_v, sidx_v, didx_v):
    c = lax.axis_index("c")
    s = lax.axis_index("s")
    wid = c * NS + s
    pltpu.sync_copy(el_hbm, el_v)
    pltpu.sync_copy(er_hbm, er_v)
    _zero_rows(tp_v, 16)
    iota16 = lax.iota(I32, 16)

    def chunk(cix, _):
        e0 = wid * EPW + cix * CH
        pltpu.sync_copy(src_hbm.at[pl.ds(e0, CH)], sidx_v)
        pltpu.sync_copy(dst_hbm.at[pl.ds(e0, CH)], didx_v)

        def grp(g, _):
            s16 = sidx_v[pl.ds(g * 16, 16)]
            d16 = didx_v[pl.ds(g * 16, 16)]
            row_ix = g * 16 + iota16
            for h in range(HEADS):
                a = plsc.load_gather(el_v, [s16 * HEADS + h])
                b = plsc.load_gather(er_v, [d16 * HEADS + h])
                x = a + b
                x = jnp.where(x >= 0.0, x, 0.2 * x)
                t = jnp.exp(x)
                plsc.store_scatter(tp_v, [row_ix, jnp.full((16,), h, I32)], t)
            return 0
        lax.fori_loop(0, CH // 16, grp, 0)
        pltpu.sync_copy(tp_v, t_hbm.at[pl.ds(e0, CH)])
        return 0
    lax.fori_loop(0, NCH, chunk, 0)


_t_kernel = pl.kernel(
    _t_body,
    out_type=jax.ShapeDtypeStruct((E, 16), F32),
    mesh=_MESH,
    compiler_params=_SC_PARAMS,
    scratch_types=[
        pltpu.VMEM((N * HEADS,), F32),
        pltpu.VMEM((N * HEADS,), F32),
        pltpu.VMEM((CH, 16), F32),
        pltpu.VMEM((CH,), I32),
        pltpu.VMEM((CH,), I32),
    ],
)


def _den_body(t_hbm, dst_hbm, out_hbm, rows_v, aux_v, didx_v, acc_sh):
    c = lax.axis_index("c")
    s = lax.axis_index("s")
    wid = c * NS + s
    iota16 = lax.iota(I32, 16)
    zero16 = jnp.zeros((16,), F32)

    def zrow(i, _):
        iv = jnp.full((16,), i, I32)
        for f in range(H // 16):
            plsc.store_scatter(rows_v, [iv, f * 16 + iota16], zero16)
        return 0
    lax.fori_loop(0, CH, zrow, 0)
    _zero_acc(rows_v, acc_sh, s)
    plsc.subcore_barrier()

    def chunk(cix, _):
        e0 = wid * EPW + cix * CH
        pltpu.sync_copy(dst_hbm.at[pl.ds(e0, CH)], didx_v)
        pltpu.sync_copy(t_hbm.at[pl.ds(e0, CH)], aux_v)

        def grp(g, _):
            row_ix = g * 16 + iota16
            for h in range(HEADS):
                hv = jnp.full((16,), h, I32)
                t = plsc.load_gather(aux_v, [row_ix, hv])
                plsc.store_scatter(rows_v, [row_ix, hv], t)
            return 0
        lax.fori_loop(0, CH // 16, grp, 0)
        pltpu.sync_copy(rows_v, acc_sh.at[didx_v], add=True)
        return 0
    lax.fori_loop(0, NCH, chunk, 0)
    plsc.subcore_barrier()

    r0 = s * SR
    pltpu.sync_copy(acc_sh.at[pl.ds(r0, SR)], out_hbm.at[pl.ds(c * NP + r0, SR)])


_den_kernel = pl.kernel(
    _den_body,
    out_type=jax.ShapeDtypeStruct((NC * NP, H), F32),
    mesh=_MESH,
    compiler_params=_SC_PARAMS,
    scratch_types=[
        pltpu.VMEM((CH, H), F32),
        pltpu.VMEM((CH, 16), F32),
        pltpu.VMEM((CH,), I32),
        pltpu.VMEM_SHARED((NP, H), F32),
    ],
)


def _wseg_body(head, x_hbm, src_hbm, dst_hbm, t_hbm, out_hbm,
               rows_v, aux_v, sidx_v, didx_v, acc_sh, sem):
    c = lax.axis_index("c")
    s = lax.axis_index("s")
    wid = c * NS + s
    hcol = jnp.full((16,), head, I32)
    iota16 = lax.iota(I32, 16)
    zero16 = jnp.zeros((16,), F32)

    def zrow(i, _):
        iv = jnp.full((16,), i, I32)
        for f in range(H // 16):
            plsc.store_scatter(rows_v, [iv, f * 16 + iota16], zero16)
        return 0
    lax.fori_loop(0, CH, zrow, 0)
    _zero_acc(rows_v, acc_sh, s)
    plsc.subcore_barrier()

    def chunk(cix, _):
        e0 = wid * EPW + cix * CH
        pltpu.sync_copy(src_hbm.at[pl.ds(e0, CH)], sidx_v)
        pltpu.sync_copy(dst_hbm.at[pl.ds(e0, CH)], didx_v)
        pltpu.sync_copy(t_hbm.at[pl.ds(e0, CH)], aux_v)
        pltpu.async_copy(x_hbm.at[sidx_v], rows_v, sem).wait()

        def medge(i, _):
            iv = jnp.full((16,), i, I32)
            w = plsc.load_gather(aux_v, [iv, hcol])
            for f in range(H // 16):
                col = f * 16 + iota16
                v = plsc.load_gather(rows_v, [iv, col]) * w
                plsc.store_scatter(rows_v, [iv, col], v)
            return 0
        lax.fori_loop(0, CH, medge, 0)
        pltpu.sync_copy(rows_v, acc_sh.at[didx_v], add=True)
        return 0
    lax.fori_loop(0, NCH, chunk, 0)
    plsc.subcore_barrier()

    r0 = s * SR
    pltpu.sync_copy(acc_sh.at[pl.ds(r0, SR)], out_hbm.at[pl.ds(c * NP + r0, SR)])


def _make_wseg(head):
    return pl.kernel(
        functools.partial(_wseg_body, head),
        out_type=jax.ShapeDtypeStruct((NC * NP, H), F32),
        mesh=_MESH,
        compiler_params=_SC_PARAMS,
        scratch_types=[
            pltpu.VMEM((CH, H), F32),
            pltpu.VMEM((CH, 16), F32),
            pltpu.VMEM((CH,), I32),
            pltpu.VMEM((CH,), I32),
            pltpu.VMEM_SHARED((NP, H), F32),
            pltpu.SemaphoreType.DMA,
        ],
    )


_wseg = [_make_wseg(h) for h in range(HEADS)]


def _embed_body(x_ref, w_ref, b_ref, o_ref):
    o_ref[...] = (jnp.dot(x_ref[...], w_ref[...], preferred_element_type=F32)
                  + b_ref[...])


def _embed(x, w, b):
    return pl.pallas_call(
        _embed_body,
        grid=(NB,),
        in_specs=[
            pl.BlockSpec((BN, H), lambda i: (i, 0)),
            pl.BlockSpec((H, H), lambda i: (0, 0)),
            pl.BlockSpec((1, H), lambda i: (0, 0)),
        ],
        out_specs=pl.BlockSpec((BN, H), lambda i: (i, 0)),
        out_shape=jax.ShapeDtypeStruct((N, H), F32),
    )(x, w, b)


def _sage_body(h_ref, p_ref, degp_ref, ws_ref, wn_ref, b_ref, o_ref):
    deg = degp_ref[0, :, 0:1] + degp_ref[1, :, 0:1]
    neigh = (p_ref[0] + p_ref[1]) / jnp.maximum(deg, 1.0)
    y = (jnp.dot(h_ref[...], ws_ref[...], preferred_element_type=F32)
         + jnp.dot(neigh, wn_ref[...], preferred_element_type=F32)
         + b_ref[...])
    o_ref[...] = jnp.maximum(y, 0.0)


def _sage(h, p, degp, ws, wn, b):
    return pl.pallas_call(
        _sage_body,
        grid=(NB,),
        in_specs=[
            pl.BlockSpec((BN, H), lambda i: (i, 0)),
            pl.BlockSpec((NC, BN, H), lambda i: (0, i, 0)),
            pl.BlockSpec((NC, BN, H), lambda i: (0, i, 0)),
            pl.BlockSpec((H, H), lambda i: (0, 0)),
            pl.BlockSpec((H, H), lambda i: (0, 0)),
            pl.BlockSpec((1, H), lambda i: (0, 0)),
        ],
        out_specs=pl.BlockSpec((BN, H), lambda i: (i, 0)),
        out_shape=jax.ShapeDtypeStruct((N, H), F32),
    )(h, p, degp, ws, wn, b)


def _attn_body(h_ref, wg_ref, al_ref, ar_ref, el_ref, er_ref):
    ft = jnp.dot(h_ref[...], wg_ref[...], preferred_element_type=F32)
    el_cols = []
    er_cols = []
    for h in range(HEADS):
        fth = ft[:, h * H:(h + 1) * H]
        el_cols.append(jnp.sum(fth * al_ref[h, :][None, :], axis=1)[:, None])
        er_cols.append(jnp.sum(fth * ar_ref[h, :][None, :], axis=1)[:, None])
    el_ref[...] = jnp.concatenate(el_cols, axis=1)
    er_ref[...] = jnp.concatenate(er_cols, axis=1)


def _attn(h, wg, al, ar):
    return pl.pallas_call(
        _attn_body,
        grid=(NB,),
        in_specs=[
            pl.BlockSpec((BN, H), lambda i: (i, 0)),
            pl.BlockSpec((H, HEADS * H), lambda i: (0, 0)),
            pl.BlockSpec((HEADS, H), lambda i: (0, 0)),
            pl.BlockSpec((HEADS, H), lambda i: (0, 0)),
        ],
        out_specs=[
            pl.BlockSpec((BN, HEADS), lambda i: (i, 0)),
            pl.BlockSpec((BN, HEADS), lambda i: (i, 0)),
        ],
        out_shape=[jax.ShapeDtypeStruct((N, HEADS), F32),
                   jax.ShapeDtypeStruct((N, HEADS), F32)],
    )(h, wg, al, ar)


def _final_body(s0_ref, s1_ref, s2_ref, s3_ref, den_ref, wg_ref, bg_ref,
                w1_ref, b1_ref, w2_ref, b2_ref, o_ref):
    s_refs = (s0_ref, s1_ref, s2_ref, s3_ref)
    parts = []
    for h in range(HEADS):
        den = den_ref[0, :, h:h + 1] + den_ref[1, :, h:h + 1]
        z = (s_refs[h][0] + s_refs[h][1]) / (den + 1e-9)
        parts.append(jnp.dot(z, wg_ref[:, h * H:(h + 1) * H],
                             preferred_element_type=F32))
    u = jnp.concatenate(parts, axis=1) + bg_ref[...]
    v = jnp.maximum(jnp.dot(u, w1_ref[...], preferred_element_type=F32)
                    + b1_ref[...], 0.0)
    o_ref[...] = (jnp.dot(v, w2_ref[...], preferred_element_type=F32)
                  + b2_ref[...])


def _final(s, den, wg, bg, w1, b1, w2, b2):
    return pl.pallas_call(
        _final_body,
        grid=(NB,),
        in_specs=[
            pl.BlockSpec((NC, BN, H), lambda i: (0, i, 0)),
            pl.BlockSpec((NC, BN, H), lambda i: (0, i, 0)),
            pl.BlockSpec((NC, BN, H), lambda i: (0, i, 0)),
            pl.BlockSpec((NC, BN, H), lambda i: (0, i, 0)),
            pl.BlockSpec((NC, BN, H), lambda i: (0, i, 0)),
            pl.BlockSpec((H, HEADS * H), lambda i: (0, 0)),
            pl.BlockSpec((1, HEADS * H), lambda i: (0, 0)),
            pl.BlockSpec((HEADS * H, H), lambda i: (0, 0)),
            pl.BlockSpec((1, H), lambda i: (0, 0)),
            pl.BlockSpec((H, C), lambda i: (0, 0)),
            pl.BlockSpec((1, C), lambda i: (0, 0)),
        ],
        out_specs=pl.BlockSpec((BN, C), lambda i: (i, 0)),
        out_shape=jax.ShapeDtypeStruct((N, C), F32),
    )(s[0], s[1], s[2], s[3], den, wg, bg, w1, b1, w2, b2)


@jax.jit
def _run(in_feat, edge_index, W_embed, b_embed, W_self1, W_neigh1, b1,
         W_self2, W_neigh2, b2, W_gat, attn_l, attn_r, b_gat,
         W_fc1, b_fc1, W_fc2, b_fc2):
    src = edge_index[0]
    dst = edge_index[1]
    h0 = _embed(in_feat, W_embed, b_embed.reshape(1, H))
    degp = _deg_kernel(dst).reshape(NC, NP, H)[:, :N]
    p1 = _seg_sum(h0, src, dst).reshape(NC, NP, H)[:, :N]
    h1 = _sage(h0, p1, degp, W_self1, W_neigh1, b1.reshape(1, H))
    p2 = _seg_sum(h1, src, dst).reshape(NC, NP, H)[:, :N]
    h2 = _sage(h1, p2, degp, W_self2, W_neigh2, b2.reshape(1, H))
    el, er = _attn(h2, W_gat, attn_l, attn_r)
    t16 = _t_kernel(el.reshape(-1), er.reshape(-1), src, dst)
    den = _den_kernel(t16, dst).reshape(NC, NP, H)[:, :N]
    s = [_wseg[h](h2, src, dst, t16).reshape(NC, NP, H)[:, :N]
         for h in range(HEADS)]
    return _final(s, den, W_gat, b_gat.reshape(1, HEADS * H),
                  W_fc1, b_fc1.reshape(1, H), W_fc2, b_fc2.reshape(1, C))


def kernel(in_feat, edge_index, W_embed, b_embed, W_self1, W_neigh1, b1,
           W_self2, W_neigh2, b2, W_gat, attn_l, attn_r, b_gat,
           W_fc1, b_fc1, W_fc2, b_fc2):
    return _run(in_feat, edge_index, W_embed, b_embed, W_self1, W_neigh1, b1,
                W_self2, W_neigh2, b2, W_gat, attn_l, attn_r, b_gat,
                W_fc1, b_fc1, W_fc2, b_fc2)


# speedup vs baseline: 9.2506x; 1.0577x over previous
import functools

import jax
import jax.numpy as jnp
from jax import lax
from jax.experimental import pallas as pl
from jax.experimental.pallas import tpu as pltpu
from jax.experimental.pallas import tpu_sc as plsc

N = 10000
E = 320000
H = 128
HEADS = 4
C = 40

NC = 2
NS = 16
NW = NC * NS
EPW = E // NW
CH = 80
NCH = EPW // CH
NP = 10240
SR = NP // NS
F32 = jnp.float32
I32 = jnp.int32

BN = 2000
NB = N // BN

_MESH = plsc.VectorSubcoreMesh(core_axis_name="c", subcore_axis_name="s",
                               num_cores=NC, num_subcores=NS)
_SC_PARAMS = pltpu.CompilerParams(needs_layout_passes=False)


def _zero_rows(rows_v, width):
    zero16 = jnp.zeros((16,), F32)

    def zrow(i, _):
        for f in range(width // 16):
            rows_v[i, pl.ds(f * 16, 16)] = zero16
        return 0
    lax.fori_loop(0, CH, zrow, 0)


def _zero_acc(rows_v, acc_sh, s):
    for k in range(SR // CH):
        pltpu.sync_copy(rows_v, acc_sh.at[pl.ds(s * SR + k * CH, CH)])


def _seg_body(x_hbm, src_hbm, dst_hbm, out_hbm,
              rows_v, rows2_v, sidx_v, sidx2_v, didx_v, didx2_v,
              acc_sh, sem, sem2):
    c = lax.axis_index("c")
    s = lax.axis_index("s")
    wid = c * NS + s
    _zero_rows(rows_v, H)
    _zero_acc(rows_v, acc_sh, s)
    plsc.subcore_barrier()

    def pair(k, _):
        e0 = wid * EPW + (2 * k) * CH
        pltpu.sync_copy(src_hbm.at[pl.ds(e0, CH)], sidx_v)
        pltpu.sync_copy(dst_hbm.at[pl.ds(e0, CH)], didx_v)
        pltpu.sync_copy(src_hbm.at[pl.ds(e0 + CH, CH)], sidx2_v)
        pltpu.sync_copy(dst_hbm.at[pl.ds(e0 + CH, CH)], didx2_v)
        cp_a = pltpu.async_copy(x_hbm.at[sidx_v], rows_v, sem)
        cp_b = pltpu.async_copy(x_hbm.at[sidx2_v], rows2_v, sem2)
        cp_a.wait()
        pltpu.sync_copy(rows_v, acc_sh.at[didx_v], add=True)
        cp_b.wait()
        pltpu.sync_copy(rows2_v, acc_sh.at[didx2_v], add=True)
        return 0
    lax.fori_loop(0, NCH // 2, pair, 0)
    if NCH % 2:
        e0 = wid * EPW + (NCH - 1) * CH
        pltpu.sync_copy(src_hbm.at[pl.ds(e0, CH)], sidx_v)
        pltpu.sync_copy(dst_hbm.at[pl.ds(e0, CH)], didx_v)
        pltpu.async_copy(x_hbm.at[sidx_v], rows_v, sem).wait()
        pltpu.sync_copy(rows_v, acc_sh.at[didx_v], add=True)
    plsc.subcore_barrier()

    r0 = s * SR
    pltpu.sync_copy(acc_sh.at[pl.ds(r0, SR)], out_hbm.at[pl.ds(c * NP + r0, SR)])


_seg_sum = pl.kernel(
    _seg_body,
    out_type=jax.ShapeDtypeStruct((NC * NP, H), F32),
    mesh=_MESH,
    scratch_types=[
        pltpu.VMEM((CH, H), F32),
        pltpu.VMEM((CH, H), F32),
        pltpu.VMEM((CH,), I32),
        pltpu.VMEM((CH,), I32),
        pltpu.VMEM((CH,), I32),
        pltpu.VMEM((CH,), I32),
        pltpu.VMEM_SHARED((NP, H), F32),
        pltpu.SemaphoreType.DMA,
        pltpu.SemaphoreType.DMA,
    ],
)


def _deg_body(dst_hbm, out_hbm, rows_v, didx_v, acc_sh):
    c = lax.axis_index("c")
    s = lax.axis_index("s")
    wid = c * NS + s
    _zero_rows(rows_v, H)
    _zero_acc(rows_v, acc_sh, s)
    one16 = jnp.ones((16,), F32)

    def orow(i, _):
        rows_v[i, pl.ds(0, 16)] = one16
        return 0
    lax.fori_loop(0, CH, orow, 0)
    plsc.subcore_barrier()

    def chunk(cix, _):
        e0 = wid * EPW + cix * CH
        pltpu.sync_copy(dst_hbm.at[pl.ds(e0, CH)], didx_v)
        pltpu.sync_copy(rows_v, acc_sh.at[didx_v], add=True)
        return 0
    lax.fori_loop(0, NCH, chunk, 0)
    plsc.subcore_barrier()

    r0 = s * SR
    pltpu.sync_copy(acc_sh.at[pl.ds(r0, SR)], out_hbm.at[pl.ds(c * NP + r0, SR)])


_deg_kernel = pl.kernel(
    _deg_body,
    out_type=jax.ShapeDtypeStruct((NC * NP, H), F32),
    mesh=_MESH,
    scratch_types=[
        pltpu.VMEM((CH, H), F32),
        pltpu.VMEM((CH,), I32),
        pltpu.VMEM_SHARED((NP, H), F32),
    ],
)


def _t_body(el_hbm, er_hbm, src_hbm, dst_hbm, t_hbm,
            el_v, er_v, tp_v, sidx_v, didx_v):
    c = lax.axis_index("c")
    s = lax.axis_index("s")
    wid = c * NS + s
    pltpu.sync_copy(el_hbm, el_v)
    pltpu.sync_copy(er_hbm, er_v)
    _zero_rows(tp_v, 16)
    iota16 = lax.iota(I32, 16)

    def chunk(cix, _):
        e0 = wid * EPW + cix * CH
        pltpu.sync_copy(src_hbm.at[pl.ds(e0, CH)], sidx_v)
        pltpu.sync_copy(dst_hbm.at[pl.ds(e0, CH)], didx_v)

        def grp(g, _):
            s16 = sidx_v[pl.ds(g * 16, 16)]
            d16 = didx_v[pl.ds(g * 16, 16)]
            row_ix = g * 16 + iota16
            for h in range(HEADS):
                a = plsc.load_gather(el_v, [s16 * HEADS + h])
                b = plsc.load_gather(er_v, [d16 * HEADS + h])
                x = a + b
                x = jnp.where(x >= 0.0, x, 0.2 * x)
                t = jnp.exp(x)
                plsc.store_scatter(tp_v, [row_ix, jnp.full((16,), h, I32)], t)
            return 0
        lax.fori_loop(0, CH // 16, grp, 0)
        pltpu.sync_copy(tp_v, t_hbm.at[pl.ds(e0, CH)])
        return 0
    lax.fori_loop(0, NCH, chunk, 0)


_t_kernel = pl.kernel(
    _t_body,
    out_type=jax.ShapeDtypeStruct((E, 16), F32),
    mesh=_MESH,
    compiler_params=_SC_PARAMS,
    scratch_types=[
        pltpu.VMEM((N * HEADS,), F32),
        pltpu.VMEM((N * HEADS,), F32),
        pltpu.VMEM((CH, 16), F32),
        pltpu.VMEM((CH,), I32),
        pltpu.VMEM((CH,), I32),
    ],
)


def _den_body(t_hbm, dst_hbm, out_hbm, rows_v, aux_v, didx_v, acc_sh):
    c = lax.axis_index("c")
    s = lax.axis_index("s")
    wid = c * NS + s
    iota16 = lax.iota(I32, 16)
    zero16 = jnp.zeros((16,), F32)

    def zrow(i, _):
        iv = jnp.full((16,), i, I32)
        for f in range(H // 16):
            plsc.store_scatter(rows_v, [iv, f * 16 + iota16], zero16)
        return 0
    lax.fori_loop(0, CH, zrow, 0)
    _zero_acc(rows_v, acc_sh, s)
    plsc.subcore_barrier()

    def chunk(cix, _):
        e0 = wid * EPW + cix * CH
        pltpu.sync_copy(dst_hbm.at[pl.ds(e0, CH)], didx_v)
        pltpu.sync_copy(t_hbm.at[pl.ds(e0, CH)], aux_v)

        def grp(g, _):
            row_ix = g * 16 + iota16
            for h in range(HEADS):
                hv = jnp.full((16,), h, I32)
                t = plsc.load_gather(aux_v, [row_ix, hv])
                plsc.store_scatter(rows_v, [row_ix, hv], t)
            return 0
        lax.fori_loop(0, CH // 16, grp, 0)
        pltpu.sync_copy(rows_v, acc_sh.at[didx_v], add=True)
        return 0
    lax.fori_loop(0, NCH, chunk, 0)
    plsc.subcore_barrier()

    r0 = s * SR
    pltpu.sync_copy(acc_sh.at[pl.ds(r0, SR)], out_hbm.at[pl.ds(c * NP + r0, SR)])


_den_kernel = pl.kernel(
    _den_body,
    out_type=jax.ShapeDtypeStruct((NC * NP, H), F32),
    mesh=_MESH,
    compiler_params=_SC_PARAMS,
    scratch_types=[
        pltpu.VMEM((CH, H), F32),
        pltpu.VMEM((CH, 16), F32),
        pltpu.VMEM((CH,), I32),
        pltpu.VMEM_SHARED((NP, H), F32),
    ],
)


def _wseg_body(head, x_hbm, src_hbm, dst_hbm, t_hbm, out_hbm,
               rows_v, rows2_v, aux_v, aux2_v, sidx_v, sidx2_v,
               didx_v, didx2_v, acc_sh, sem, sem2):
    c = lax.axis_index("c")
    s = lax.axis_index("s")
    wid = c * NS + s
    hcol = jnp.full((16,), head, I32)
    iota16 = lax.iota(I32, 16)
    zero16 = jnp.zeros((16,), F32)

    def zrow(i, _):
        iv = jnp.full((16,), i, I32)
        for f in range(H // 16):
            plsc.store_scatter(rows_v, [iv, f * 16 + iota16], zero16)
        return 0
    lax.fori_loop(0, CH, zrow, 0)
    _zero_acc(rows_v, acc_sh, s)
    plsc.subcore_barrier()

    def _mul_scat(rows, aux, didx):
        def medge(i, _):
            iv = jnp.full((16,), i, I32)
            w = plsc.load_gather(aux, [iv, hcol])
            for f in range(H // 16):
                col = f * 16 + iota16
                v = plsc.load_gather(rows, [iv, col]) * w
                plsc.store_scatter(rows, [iv, col], v)
            return 0
        lax.fori_loop(0, CH, medge, 0)
        pltpu.sync_copy(rows, acc_sh.at[didx], add=True)

    def pair(k, _):
        e0 = wid * EPW + (2 * k) * CH
        pltpu.sync_copy(src_hbm.at[pl.ds(e0, CH)], sidx_v)
        pltpu.sync_copy(dst_hbm.at[pl.ds(e0, CH)], didx_v)
        pltpu.sync_copy(t_hbm.at[pl.ds(e0, CH)], aux_v)
        pltpu.sync_copy(src_hbm.at[pl.ds(e0 + CH, CH)], sidx2_v)
        pltpu.sync_copy(dst_hbm.at[pl.ds(e0 + CH, CH)], didx2_v)
        pltpu.sync_copy(t_hbm.at[pl.ds(e0 + CH, CH)], aux2_v)
        cp_a = pltpu.async_copy(x_hbm.at[sidx_v], rows_v, sem)
        cp_b = pltpu.async_copy(x_hbm.at[sidx2_v], rows2_v, sem2)
        cp_a.wait()
        _mul_scat(rows_v, aux_v, didx_v)
        cp_b.wait()
        _mul_scat(rows2_v, aux2_v, didx2_v)
        return 0
    lax.fori_loop(0, NCH // 2, pair, 0)
    if NCH % 2:
        e0 = wid * EPW + (NCH - 1) * CH
        pltpu.sync_copy(src_hbm.at[pl.ds(e0, CH)], sidx_v)
        pltpu.sync_copy(dst_hbm.at[pl.ds(e0, CH)], didx_v)
        pltpu.sync_copy(t_hbm.at[pl.ds(e0, CH)], aux_v)
        pltpu.async_copy(x_hbm.at[sidx_v], rows_v, sem).wait()
        _mul_scat(rows_v, aux_v, didx_v)
    plsc.subcore_barrier()

    r0 = s * SR
    pltpu.sync_copy(acc_sh.at[pl.ds(r0, SR)], out_hbm.at[pl.ds(c * NP + r0, SR)])


def _make_wseg(head):
    return pl.kernel(
        functools.partial(_wseg_body, head),
        out_type=jax.ShapeDtypeStruct((NC * NP, H), F32),
        mesh=_MESH,
        compiler_params=_SC_PARAMS,
        scratch_types=[
            pltpu.VMEM((CH, H), F32),
            pltpu.VMEM((CH, H), F32),
            pltpu.VMEM((CH, 16), F32),
            pltpu.VMEM((CH, 16), F32),
            pltpu.VMEM((CH,), I32),
            pltpu.VMEM((CH,), I32),
            pltpu.VMEM((CH,), I32),
            pltpu.VMEM((CH,), I32),
            pltpu.VMEM_SHARED((NP, H), F32),
            pltpu.SemaphoreType.DMA,
            pltpu.SemaphoreType.DMA,
        ],
    )


_wseg = [_make_wseg(h) for h in range(HEADS)]


def _embed_body(x_ref, w_ref, b_ref, o_ref):
    o_ref[...] = (jnp.dot(x_ref[...], w_ref[...], preferred_element_type=F32)
                  + b_ref[...])


def _embed(x, w, b):
    return pl.pallas_call(
        _embed_body,
        grid=(NB,),
        in_specs=[
            pl.BlockSpec((BN, H), lambda i: (i, 0)),
            pl.BlockSpec((H, H), lambda i: (0, 0)),
            pl.BlockSpec((1, H), lambda i: (0, 0)),
        ],
        out_specs=pl.BlockSpec((BN, H), lambda i: (i, 0)),
        out_shape=jax.ShapeDtypeStruct((N, H), F32),
    )(x, w, b)


def _sage_body(h_ref, p_ref, degp_ref, ws_ref, wn_ref, b_ref, o_ref):
    deg = degp_ref[0, :, 0:1] + degp_ref[1, :, 0:1]
    neigh = (p_ref[0] + p_ref[1]) / jnp.maximum(deg, 1.0)
    y = (jnp.dot(h_ref[...], ws_ref[...], preferred_element_type=F32)
         + jnp.dot(neigh, wn_ref[...], preferred_element_type=F32)
         + b_ref[...])
    o_ref[...] = jnp.maximum(y, 0.0)


def _sage(h, p, degp, ws, wn, b):
    return pl.pallas_call(
        _sage_body,
        grid=(NB,),
        in_specs=[
            pl.BlockSpec((BN, H), lambda i: (i, 0)),
            pl.BlockSpec((NC, BN, H), lambda i: (0, i, 0)),
            pl.BlockSpec((NC, BN, H), lambda i: (0, i, 0)),
            pl.BlockSpec((H, H), lambda i: (0, 0)),
            pl.BlockSpec((H, H), lambda i: (0, 0)),
            pl.BlockSpec((1, H), lambda i: (0, 0)),
        ],
        out_specs=pl.BlockSpec((BN, H), lambda i: (i, 0)),
        out_shape=jax.ShapeDtypeStruct((N, H), F32),
    )(h, p, degp, ws, wn, b)


def _attn_body(h_ref, wg_ref, al_ref, ar_ref, el_ref, er_ref):
    ft = jnp.dot(h_ref[...], wg_ref[...], preferred_element_type=F32)
    el_cols = []
    er_cols = []
    for h in range(HEADS):
        fth = ft[:, h * H:(h + 1) * H]
        el_cols.append(jnp.sum(fth * al_ref[h, :][None, :], axis=1)[:, None])
        er_cols.append(jnp.sum(fth * ar_ref[h, :][None, :], axis=1)[:, None])
    el_ref[...] = jnp.concatenate(el_cols, axis=1)
    er_ref[...] = jnp.concatenate(er_cols, axis=1)


def _attn(h, wg, al, ar):
    return pl.pallas_call(
        _attn_body,
        grid=(NB,),
        in_specs=[
            pl.BlockSpec((BN, H), lambda i: (i, 0)),
            pl.BlockSpec((H, HEADS * H), lambda i: (0, 0)),
            pl.BlockSpec((HEADS, H), lambda i: (0, 0)),
            pl.BlockSpec((HEADS, H), lambda i: (0, 0)),
        ],
        out_specs=[
            pl.BlockSpec((BN, HEADS), lambda i: (i, 0)),
            pl.BlockSpec((BN, HEADS), lambda i: (i, 0)),
        ],
        out_shape=[jax.ShapeDtypeStruct((N, HEADS), F32),
                   jax.ShapeDtypeStruct((N, HEADS), F32)],
    )(h, wg, al, ar)


def _final_body(s0_ref, s1_ref, s2_ref, s3_ref, den_ref, wg_ref, bg_ref,
                w1_ref, b1_ref, w2_ref, b2_ref, o_ref):
    s_refs = (s0_ref, s1_ref, s2_ref, s3_ref)
    parts = []
    for h in range(HEADS):
        den = den_ref[0, :, h:h + 1] + den_ref[1, :, h:h + 1]
        z = (s_refs[h][0] + s_refs[h][1]) / (den + 1e-9)
        parts.append(jnp.dot(z, wg_ref[:, h * H:(h + 1) * H],
                             preferred_element_type=F32))
    u = jnp.concatenate(parts, axis=1) + bg_ref[...]
    v = jnp.maximum(jnp.dot(u, w1_ref[...], preferred_element_type=F32)
                    + b1_ref[...], 0.0)
    o_ref[...] = (jnp.dot(v, w2_ref[...], preferred_element_type=F32)
                  + b2_ref[...])


def _final(s, den, wg, bg, w1, b1, w2, b2):
    return pl.pallas_call(
        _final_body,
        grid=(NB,),
        in_specs=[
            pl.BlockSpec((NC, BN, H), lambda i: (0, i, 0)),
            pl.BlockSpec((NC, BN, H), lambda i: (0, i, 0)),
            pl.BlockSpec((NC, BN, H), lambda i: (0, i, 0)),
            pl.BlockSpec((NC, BN, H), lambda i: (0, i, 0)),
            pl.BlockSpec((NC, BN, H), lambda i: (0, i, 0)),
            pl.BlockSpec((H, HEADS * H), lambda i: (0, 0)),
            pl.BlockSpec((1, HEADS * H), lambda i: (0, 0)),
            pl.BlockSpec((HEADS * H, H), lambda i: (0, 0)),
            pl.BlockSpec((1, H), lambda i: (0, 0)),
            pl.BlockSpec((H, C), lambda i: (0, 0)),
            pl.BlockSpec((1, C), lambda i: (0, 0)),
        ],
        out_specs=pl.BlockSpec((BN, C), lambda i: (i, 0)),
        out_shape=jax.ShapeDtypeStruct((N, C), F32),
    )(s[0], s[1], s[2], s[3], den, wg, bg, w1, b1, w2, b2)


@jax.jit
def _run(in_feat, edge_index, W_embed, b_embed, W_self1, W_neigh1, b1,
         W_self2, W_neigh2, b2, W_gat, attn_l, attn_r, b_gat,
         W_fc1, b_fc1, W_fc2, b_fc2):
    src = edge_index[0]
    dst = edge_index[1]
    h0 = _embed(in_feat, W_embed, b_embed.reshape(1, H))
    degp = _deg_kernel(dst).reshape(NC, NP, H)[:, :N]
    p1 = _seg_sum(h0, src, dst).reshape(NC, NP, H)[:, :N]
    h1 = _sage(h0, p1, degp, W_self1, W_neigh1, b1.reshape(1, H))
    p2 = _seg_sum(h1, src, dst).reshape(NC, NP, H)[:, :N]
    h2 = _sage(h1, p2, degp, W_self2, W_neigh2, b2.reshape(1, H))
    el, er = _attn(h2, W_gat, attn_l, attn_r)
    t16 = _t_kernel(el.reshape(-1), er.reshape(-1), src, dst)
    den = _den_kernel(t16, dst).reshape(NC, NP, H)[:, :N]
    s = [_wseg[h](h2, src, dst, t16).reshape(NC, NP, H)[:, :N]
         for h in range(HEADS)]
    return _final(s, den, W_gat, b_gat.reshape(1, HEADS * H),
                  W_fc1, b_fc1.reshape(1, H), W_fc2, b_fc2.reshape(1, C))


def kernel(in_feat, edge_index, W_embed, b_embed, W_self1, W_neigh1, b1,
           W_self2, W_neigh2, b2, W_gat, attn_l, attn_r, b_gat,
           W_fc1, b_fc1, W_fc2, b_fc2):
    return _run(in_feat, edge_index, W_embed, b_embed, W_self1, W_neigh1, b1,
                W_self2, W_neigh2, b2, W_gat, attn_l, attn_r, b_gat,
                W_fc1, b_fc1, W_fc2, b_fc2)
